# Initial kernel scaffold; baseline (speedup 1.0000x reference)
import jax
import jax.numpy as jnp
from jax.experimental import pallas as pl


def kernel(hidden, input_ids, W, b_lin, lag_logits, copy_scale):
    raise NotImplementedError("write your pallas kernel here")


# speedup vs baseline: 6.3567x; 6.3567x over previous
import functools

import jax
import jax.numpy as jnp
from jax import lax
from jax.experimental import pallas as pl
from jax.experimental.pallas import tpu as pltpu

T_BLOCK = 32


def _bias_block_kernel(toks_ref, lw_ref, hidden_ref, w_ref, b_ref, out_ref,
                       *, window, vocab, t_block):
    h = hidden_ref[...]
    w = w_ref[...]
    logits = jnp.dot(h, w, preferred_element_type=jnp.float32)
    g = jax.nn.sigmoid(logits + b_ref[...][0, 0])[:, 0:1]
    cols = lax.broadcasted_iota(jnp.int32, (1, vocab), 1)
    acc = jnp.zeros((t_block, vocab), jnp.float32)
    toks = toks_ref[...]
    for l in range(window):
        tok = toks[:, l].reshape(t_block, 1)
        contrib = g * lw_ref[...][0, l]
        acc = acc + jnp.where(tok == cols, contrib, 0.0)
    out_ref[...] = acc


def kernel(hidden, input_ids, W, b_lin, lag_logits, copy_scale):
    b, t, d = hidden.shape
    vocab = 32000
    window = lag_logits.shape[0]
    ids = input_ids.reshape(t)
    toks = jnp.stack(
        [jnp.concatenate([jnp.full((l,), -1, jnp.int32), ids[: t - l]])
         for l in range(window)], axis=1)
    lw = (jax.nn.softmax(lag_logits) * copy_scale).reshape(1, window)
    h2 = hidden.reshape(t, d)
    w_pad = jnp.zeros((d, 128), jnp.float32).at[:, 0].set(W.reshape(d))
    b2 = b_lin.reshape(1, 1)

    grid = (t // T_BLOCK,)
    out = pl.pallas_call(
        functools.partial(_bias_block_kernel, window=window, vocab=vocab,
                          t_block=T_BLOCK),
        grid=grid,
        in_specs=[
            pl.BlockSpec((T_BLOCK, window), lambda i: (i, 0)),
            pl.BlockSpec((1, window), lambda i: (0, 0)),
            pl.BlockSpec((T_BLOCK, d), lambda i: (i, 0)),
            pl.BlockSpec((d, 128), lambda i: (0, 0)),
            pl.BlockSpec((1, 1), lambda i: (0, 0)),
        ],
        out_specs=pl.BlockSpec((T_BLOCK, vocab), lambda i: (i, 0)),
        out_shape=jax.ShapeDtypeStruct((t, vocab), jnp.float32),
    )(toks, lw, h2, w_pad, b2)
    return out.reshape(b, t, vocab)


# speedup vs baseline: 6.5630x; 1.0325x over previous
import functools

import jax
import jax.numpy as jnp
from jax import lax
from jax.experimental import pallas as pl
from jax.experimental.pallas import tpu as pltpu
from jax.experimental.pallas import tpu_sc as plsc

LANES = 16


def _prep_kernel(toks_ref, lw_ref, hidden_ref, w_ref, b_ref,
                 cols_ref, vals_ref, *, window, vocab):
    h = hidden_ref[...]
    w = w_ref[...]
    logits = jnp.dot(h, w, preferred_element_type=jnp.float32)
    g = jax.nn.sigmoid(logits + b_ref[...][0, 0])[:, 0:1]
    toks = toks_ref[...]
    lw = lw_ref[...]
    t = toks.shape[0]

    merged = jnp.zeros((t, window), jnp.float32)
    notfirst = jnp.zeros((t, window), jnp.bool_)
    lane = lax.broadcasted_iota(jnp.int32, (t, window), 1)
    for l2 in range(window):
        eq = toks == toks[:, l2:l2 + 1]
        merged = merged + jnp.where(eq, lw[0, l2], 0.0)
        if l2 < window - 1:
            notfirst = notfirst | (eq & (lane > l2))
    keep = (toks >= 0) & jnp.logical_not(notfirst)
    cols8 = jnp.where(keep, toks, vocab + lane)
    vals8 = jnp.where(keep, g * merged, 0.0)
    pad_cols = vocab + window + lax.broadcasted_iota(
        jnp.int32, (t, LANES - window), 1)
    cols_ref[...] = jnp.concatenate([cols8, pad_cols], axis=1)
    vals_ref[...] = jnp.concatenate(
        [vals8, jnp.zeros((t, LANES - window), jnp.float32)], axis=1)


def _make_sc_scatter(t, vocab, rows_per_w, nc, ns):
    bufw = vocab + LANES
    mesh = plsc.VectorSubcoreMesh(core_axis_name="c", subcore_axis_name="s")

    @functools.partial(
        pl.kernel,
        out_type=jax.ShapeDtypeStruct((t * vocab,), jnp.float32),
        mesh=mesh,
        scratch_types=[
            pltpu.VMEM((rows_per_w, LANES), jnp.int32),
            pltpu.VMEM((rows_per_w, LANES), jnp.float32),
            pltpu.VMEM((bufw,), jnp.float32),
            pltpu.VMEM((bufw,), jnp.float32),
            pltpu.SemaphoreType.DMA,
            pltpu.SemaphoreType.DMA,
        ],
        compiler_params=pltpu.CompilerParams(needs_layout_passes=False),
    )
    def sc_scatter(cols_hbm, vals_hbm, out_hbm, cols_v, vals_v,
                   buf0, buf1, sem0, sem1):
        wid = lax.axis_index("s") * nc + lax.axis_index("c")
        base = wid * rows_per_w
        pltpu.sync_copy(cols_hbm.at[pl.ds(base, rows_per_w)], cols_v)
        pltpu.sync_copy(vals_hbm.at[pl.ds(base, rows_per_w)], vals_v)

        z16 = jnp.zeros((LANES,), jnp.float32)

        def zero_body(i, carry):
            buf0[pl.ds(i * LANES, LANES)] = z16
            buf1[pl.ds(i * LANES, LANES)] = z16
            return carry

        lax.fori_loop(0, bufw // LANES, zero_body, 0)

        bufs = (buf0, buf1)
        sems = (sem0, sem1)
        copies = [None, None]
        for r in range(rows_per_w):
            buf = bufs[r % 2]
            if r >= 2:
                copies[r % 2].wait()
                idx_old = cols_v[r - 2, :]
                plsc.store_scatter(buf, (idx_old,), z16)
            idx = cols_v[r, :]
            val = vals_v[r, :]
            plsc.store_scatter(buf, (idx,), val)
            copies[r % 2] = pltpu.async_copy(
                buf.at[pl.ds(0, vocab)],
                out_hbm.at[pl.ds((base + r) * vocab, vocab)],
                sems[r % 2])
        copies[0].wait()
        copies[1].wait()

    return sc_scatter


def kernel(hidden, input_ids, W, b_lin, lag_logits, copy_scale):
    b, t, d = hidden.shape
    vocab = 32000
    window = lag_logits.shape[0]
    ids = input_ids.reshape(t)
    toks = jnp.stack(
        [jnp.concatenate([jnp.full((l,), -1, jnp.int32), ids[: t - l]])
         for l in range(window)], axis=1)
    lw = (jax.nn.softmax(lag_logits) * copy_scale).reshape(1, window)
    h2 = hidden.reshape(t, d)
    w_pad = jnp.zeros((d, 128), jnp.float32).at[:, 0].set(W.reshape(d))
    b2 = b_lin.reshape(1, 1)

    cols, vals = pl.pallas_call(
        functools.partial(_prep_kernel, window=window, vocab=vocab),
        out_shape=[jax.ShapeDtypeStruct((t, LANES), jnp.int32),
                   jax.ShapeDtypeStruct((t, LANES), jnp.float32)],
    )(toks, lw, h2, w_pad, b2)

    info = plsc.get_sparse_core_info()
    nc, ns = info.num_cores, info.num_subcores
    rows_per_w = t // (nc * ns)
    out_flat = _make_sc_scatter(t, vocab, rows_per_w, nc, ns)(cols, vals)
    return out_flat.reshape(b, t, vocab)


# speedup vs baseline: 15.8460x; 2.4144x over previous
import functools

import jax
import jax.numpy as jnp
from jax import lax
from jax.experimental import pallas as pl
from jax.experimental.pallas import tpu as pltpu
from jax.experimental.pallas import tpu_sc as plsc

LANES = 16


def _prep_kernel(toks_ref, lw_ref, hidden_ref, w_ref, b_ref,
                 cols_ref, vals_ref, *, window, vocab):
    h = hidden_ref[...]
    w = w_ref[...]
    logits = jnp.dot(h, w, preferred_element_type=jnp.float32)
    g = jax.nn.sigmoid(logits + b_ref[...][0, 0])[:, 0:1]
    toks = toks_ref[...]
    lw = lw_ref[...]
    t = toks.shape[0]

    merged = jnp.zeros((t, window), jnp.float32)
    notfirst = jnp.zeros((t, window), jnp.bool_)
    lane = lax.broadcasted_iota(jnp.int32, (t, window), 1)
    for l2 in range(window):
        eq = toks == toks[:, l2:l2 + 1]
        merged = merged + jnp.where(eq, lw[0, l2], 0.0)
        if l2 < window - 1:
            notfirst = notfirst | (eq & (lane > l2))
    keep = (toks >= 0) & jnp.logical_not(notfirst)
    cols8 = jnp.where(keep, toks, vocab + lane)
    vals8 = jnp.where(keep, g * merged, 0.0)
    pad_cols = vocab + window + lax.broadcasted_iota(
        jnp.int32, (t, LANES - window), 1)
    cols_ref[...] = jnp.concatenate([cols8, pad_cols], axis=1)
    vals_ref[...] = jnp.concatenate(
        [vals8, jnp.zeros((t, LANES - window), jnp.float32)], axis=1)


def _make_sc_scatter(t, vocab, rows_per_w, nc, ns):
    bufw = vocab + LANES
    mesh = plsc.VectorSubcoreMesh(core_axis_name="c", subcore_axis_name="s")

    @functools.partial(
        pl.kernel,
        out_type=jax.ShapeDtypeStruct((1, t, vocab), jnp.float32),
        mesh=mesh,
        scratch_types=[
            pltpu.VMEM((rows_per_w, LANES), jnp.int32),
            pltpu.VMEM((rows_per_w, LANES), jnp.float32),
            pltpu.VMEM((bufw,), jnp.float32),
            pltpu.VMEM((bufw,), jnp.float32),
            pltpu.SemaphoreType.DMA,
            pltpu.SemaphoreType.DMA,
        ],
        compiler_params=pltpu.CompilerParams(needs_layout_passes=False),
    )
    def sc_scatter(cols_hbm, vals_hbm, out_hbm, cols_v, vals_v,
                   buf0, buf1, sem0, sem1):
        wid = lax.axis_index("s") * nc + lax.axis_index("c")
        base = wid * rows_per_w
        pltpu.sync_copy(cols_hbm.at[pl.ds(base, rows_per_w)], cols_v)
        pltpu.sync_copy(vals_hbm.at[pl.ds(base, rows_per_w)], vals_v)

        z16 = jnp.zeros((LANES,), jnp.float32)

        def zero_body(i, carry):
            buf0[pl.ds(i * LANES, LANES)] = z16
            buf1[pl.ds(i * LANES, LANES)] = z16
            return carry

        lax.fori_loop(0, bufw // LANES, zero_body, 0)

        bufs = (buf0, buf1)
        sems = (sem0, sem1)
        copies = [None, None]
        for r in range(rows_per_w):
            buf = bufs[r % 2]
            if r >= 2:
                copies[r % 2].wait()
                idx_old = cols_v[r - 2, :]
                plsc.store_scatter(buf, (idx_old,), z16)
            idx = cols_v[r, :]
            val = vals_v[r, :]
            plsc.store_scatter(buf, (idx,), val)
            copies[r % 2] = pltpu.async_copy(
                buf.at[pl.ds(0, vocab)],
                out_hbm.at[0, base + r],
                sems[r % 2])
        copies[0].wait()
        copies[1].wait()

    return sc_scatter


def kernel(hidden, input_ids, W, b_lin, lag_logits, copy_scale):
    b, t, d = hidden.shape
    vocab = 32000
    window = lag_logits.shape[0]
    ids = input_ids.reshape(t)
    toks = jnp.stack(
        [jnp.concatenate([jnp.full((l,), -1, jnp.int32), ids[: t - l]])
         for l in range(window)], axis=1)
    lw = (jax.nn.softmax(lag_logits) * copy_scale).reshape(1, window)
    h2 = hidden.reshape(t, d)
    w_pad = jnp.zeros((d, 128), jnp.float32).at[:, 0].set(W.reshape(d))
    b2 = b_lin.reshape(1, 1)

    cols, vals = pl.pallas_call(
        functools.partial(_prep_kernel, window=window, vocab=vocab),
        out_shape=[jax.ShapeDtypeStruct((t, LANES), jnp.int32),
                   jax.ShapeDtypeStruct((t, LANES), jnp.float32)],
    )(toks, lw, h2, w_pad, b2)

    info = plsc.get_sparse_core_info()
    nc, ns = info.num_cores, info.num_subcores
    rows_per_w = t // (nc * ns)
    return _make_sc_scatter(t, vocab, rows_per_w, nc, ns)(cols, vals)


# speedup vs baseline: 17.8065x; 1.1237x over previous
import functools

import jax
import jax.numpy as jnp
from jax import lax
from jax.experimental import pallas as pl
from jax.experimental.pallas import tpu as pltpu
from jax.experimental.pallas import tpu_sc as plsc

LANES = 16
NBUF = 3


def _prep_kernel(ids_ref, lagl_ref, scale_ref, hidden_ref, w_ref, b_ref,
                 cols_ref, vals_ref, *, window, vocab):
    h = hidden_ref[...]
    w = w_ref[...]
    logits = jnp.dot(h, w, preferred_element_type=jnp.float32)
    g = jax.nn.sigmoid(logits + b_ref[...][0, 0])[:, 0:1]

    lw_row = (jax.nn.softmax(lagl_ref[...], axis=1)
              * scale_ref[...][0, 0])

    ids = ids_ref[...]
    t = ids.shape[0]
    shifted = [ids]
    for l in range(1, window):
        shifted.append(jnp.concatenate(
            [jnp.full((l, 1), -1, jnp.int32), ids[: t - l, :]], axis=0))
    toks = jnp.concatenate(shifted, axis=1)

    merged = jnp.zeros((t, window), jnp.float32)
    notfirst = jnp.zeros((t, window), jnp.bool_)
    lane = lax.broadcasted_iota(jnp.int32, (t, window), 1)
    for l2 in range(window):
        eq = toks == toks[:, l2:l2 + 1]
        merged = merged + jnp.where(eq, lw_row[0, l2], 0.0)
        if l2 < window - 1:
            notfirst = notfirst | (eq & (lane > l2))
    keep = (toks >= 0) & jnp.logical_not(notfirst)
    cols8 = jnp.where(keep, toks, vocab + lane)
    vals8 = jnp.where(keep, g * merged, 0.0)
    pad_cols = vocab + window + lax.broadcasted_iota(
        jnp.int32, (t, LANES - window), 1)
    cols_ref[...] = jnp.concatenate([cols8, pad_cols], axis=1)
    vals_ref[...] = jnp.concatenate(
        [vals8, jnp.zeros((t, LANES - window), jnp.float32)], axis=1)


def _make_sc_scatter(t, vocab, rows_per_w, nc, ns):
    bufw = vocab + 128
    zchunk = 8
    assert (bufw // LANES) % zchunk == 0
    mesh = plsc.VectorSubcoreMesh(core_axis_name="c", subcore_axis_name="s")

    @functools.partial(
        pl.kernel,
        out_type=jax.ShapeDtypeStruct((1, t, vocab), jnp.float32),
        mesh=mesh,
        scratch_types=[
            pltpu.VMEM((rows_per_w, LANES), jnp.int32),
            pltpu.VMEM((rows_per_w, LANES), jnp.float32),
        ] + [pltpu.VMEM((bufw,), jnp.float32)] * NBUF
          + [pltpu.SemaphoreType.DMA] * NBUF,
        compiler_params=pltpu.CompilerParams(needs_layout_passes=False),
    )
    def sc_scatter(cols_hbm, vals_hbm, out_hbm, cols_v, vals_v, *bufs_sems):
        bufs = bufs_sems[:NBUF]
        sems = bufs_sems[NBUF:]
        wid = lax.axis_index("s") * nc + lax.axis_index("c")
        base = wid * rows_per_w
        pltpu.sync_copy(cols_hbm.at[pl.ds(base, rows_per_w)], cols_v)
        pltpu.sync_copy(vals_hbm.at[pl.ds(base, rows_per_w)], vals_v)

        z16 = jnp.zeros((LANES,), jnp.float32)

        def zero_body(i, carry):
            for k in range(zchunk):
                off = (i * zchunk + k) * LANES
                for buf in bufs:
                    buf[pl.ds(off, LANES)] = z16
            return carry

        lax.fori_loop(0, bufw // LANES // zchunk, zero_body, 0)

        copies = [None] * NBUF
        for r in range(rows_per_w):
            buf = bufs[r % NBUF]
            if r >= NBUF:
                copies[r % NBUF].wait()
                idx_old = cols_v[r - NBUF, :]
                plsc.store_scatter(buf, (idx_old,), z16)
            idx = cols_v[r, :]
            val = vals_v[r, :]
            plsc.store_scatter(buf, (idx,), val)
            copies[r % NBUF] = pltpu.async_copy(
                buf.at[pl.ds(0, vocab)],
                out_hbm.at[0, base + r],
                sems[r % NBUF])
        for k in range(NBUF):
            copies[k].wait()

    return sc_scatter


def kernel(hidden, input_ids, W, b_lin, lag_logits, copy_scale):
    b, t, d = hidden.shape
    vocab = 32000
    window = lag_logits.shape[0]
    ids_col = input_ids.reshape(t, 1)
    lag_row = lag_logits.reshape(1, window)
    scale2 = copy_scale.reshape(1, 1)
    h2 = hidden.reshape(t, d)
    w_pad = jnp.zeros((d, 128), jnp.float32).at[:, 0].set(W.reshape(d))
    b2 = b_lin.reshape(1, 1)

    cols, vals = pl.pallas_call(
        functools.partial(_prep_kernel, window=window, vocab=vocab),
        out_shape=[jax.ShapeDtypeStruct((t, LANES), jnp.int32),
                   jax.ShapeDtypeStruct((t, LANES), jnp.float32)],
    )(ids_col, lag_row, scale2, h2, w_pad, b2)

    info = plsc.get_sparse_core_info()
    nc, ns = info.num_cores, info.num_subcores
    rows_per_w = t // (nc * ns)
    return _make_sc_scatter(t, vocab, rows_per_w, nc, ns)(cols, vals)


# speedup vs baseline: 19.0611x; 1.0705x over previous
import functools

import jax
import jax.numpy as jnp
from jax import lax
from jax.experimental import pallas as pl
from jax.experimental.pallas import tpu as pltpu
from jax.experimental.pallas import tpu_sc as plsc

LANES = 16
NBUF = 3


def _prep_kernel(ids_ref, lagl_ref, scale_ref, hidden_ref, w_ref, b_ref,
                 cols_ref, vals_ref, g_ref, *, window, vocab):
    h = hidden_ref[...][0]
    w = w_ref[...]
    logits = jnp.dot(h, w, preferred_element_type=jnp.float32)
    g = jax.nn.sigmoid(logits + b_ref[...][0, 0])[:, 0:1]
    g_ref[...] = jnp.broadcast_to(g, g_ref.shape)

    lw_row = (jax.nn.softmax(lagl_ref[...], axis=1)
              * scale_ref[...][0, 0])

    ids = ids_ref[...]
    t = ids.shape[1]
    shifted = [ids]
    for l in range(1, window):
        shifted.append(jnp.concatenate(
            [jnp.full((1, l), -1, jnp.int32), ids[:, : t - l]], axis=1))
    toks = jnp.concatenate(shifted, axis=0)

    merged = jnp.zeros((window, t), jnp.float32)
    notfirst = jnp.zeros((window, t), jnp.bool_)
    row = lax.broadcasted_iota(jnp.int32, (window, t), 0)
    for l2 in range(window):
        eq = toks == toks[l2:l2 + 1, :]
        merged = merged + jnp.where(eq, lw_row[0, l2], 0.0)
        if l2 < window - 1:
            notfirst = notfirst | (eq & (row > l2))
    keep = (toks >= 0) & jnp.logical_not(notfirst)
    cols8 = jnp.where(keep, toks, vocab + row)
    vals8 = jnp.where(keep, merged, 0.0)
    pad_cols = vocab + window + lax.broadcasted_iota(
        jnp.int32, (LANES - window, t), 0)
    cols_ref[...] = jnp.concatenate([cols8, pad_cols], axis=0)
    vals_ref[...] = jnp.concatenate(
        [vals8, jnp.zeros((LANES - window, t), jnp.float32)], axis=0)


def _make_sc_scatter(t, vocab, rows_per_w, nc, ns):
    bufw = vocab + 128
    zchunk = 8
    assert (bufw // LANES) % zchunk == 0
    mesh = plsc.VectorSubcoreMesh(core_axis_name="c", subcore_axis_name="s")

    @functools.partial(
        pl.kernel,
        out_type=jax.ShapeDtypeStruct((1, t, vocab), jnp.float32),
        mesh=mesh,
        scratch_types=[
            pltpu.VMEM((LANES, 2 * rows_per_w), jnp.int32),
            pltpu.VMEM((LANES, 2 * rows_per_w), jnp.float32),
            pltpu.VMEM((rows_per_w, 128), jnp.float32),
        ] + [pltpu.VMEM((bufw,), jnp.float32)] * NBUF
          + [pltpu.SemaphoreType.DMA] * NBUF,
        compiler_params=pltpu.CompilerParams(needs_layout_passes=False),
    )
    def sc_scatter(cols_hbm, vals_hbm, g_hbm, out_hbm, cols_v, vals_v, g_v,
                   *bufs_sems):
        bufs = bufs_sems[:NBUF]
        sems = bufs_sems[NBUF:]
        wid = lax.axis_index("s") * nc + lax.axis_index("c")
        base = wid * rows_per_w
        blk = (wid // 2) * (2 * rows_per_w)
        hoff = (wid % 2) * rows_per_w
        pltpu.sync_copy(cols_hbm.at[:, pl.ds(blk, 2 * rows_per_w)], cols_v)
        pltpu.sync_copy(vals_hbm.at[:, pl.ds(blk, 2 * rows_per_w)], vals_v)
        pltpu.sync_copy(g_hbm.at[pl.ds(base, rows_per_w)], g_v)

        z16 = jnp.zeros((LANES,), jnp.float32)
        z16i = jnp.zeros((LANES,), jnp.int32)
        iota16 = lax.broadcasted_iota(jnp.int32, (LANES,), 0)

        def zero_body(i, carry):
            for k in range(zchunk):
                off = (i * zchunk + k) * LANES
                for buf in bufs:
                    buf[pl.ds(off, LANES)] = z16
            return carry

        lax.fori_loop(0, bufw // LANES // zchunk, zero_body, 0)

        def row_vecs(r):
            rr = jnp.full((LANES,), r, jnp.int32) + hoff
            idx = plsc.load_gather(cols_v, (iota16, rr))
            return rr, idx

        copies = [None] * NBUF
        for r in range(rows_per_w):
            buf = bufs[r % NBUF]
            if r >= NBUF:
                copies[r % NBUF].wait()
                _, idx_old = row_vecs(r - NBUF)
                plsc.store_scatter(buf, (idx_old,), z16)
            rr, idx = row_vecs(r)
            vraw = plsc.load_gather(vals_v, (iota16, rr))
            gv = plsc.load_gather(g_v, (jnp.full((LANES,), r, jnp.int32), z16i))
            plsc.store_scatter(buf, (idx,), vraw * gv)
            copies[r % NBUF] = pltpu.async_copy(
                buf.at[pl.ds(0, vocab)],
                out_hbm.at[0, base + r],
                sems[r % NBUF])
        for k in range(NBUF):
            copies[k].wait()

    return sc_scatter


def kernel(hidden, input_ids, W, b_lin, lag_logits, copy_scale):
    b, t, d = hidden.shape
    vocab = 32000
    window = lag_logits.shape[0]
    lag_row = lag_logits.reshape(1, window)
    scale2 = copy_scale.reshape(1, 1)
    w_pad = jnp.zeros((d, 128), jnp.float32).at[:, 0].set(W.reshape(d))
    b2 = b_lin.reshape(1, 1)

    cols, vals, g = pl.pallas_call(
        functools.partial(_prep_kernel, window=window, vocab=vocab),
        out_shape=[jax.ShapeDtypeStruct((LANES, t), jnp.int32),
                   jax.ShapeDtypeStruct((LANES, t), jnp.float32),
                   jax.ShapeDtypeStruct((t, 128), jnp.float32)],
    )(input_ids, lag_row, scale2, hidden, w_pad, b2)

    info = plsc.get_sparse_core_info()
    nc, ns = info.num_cores, info.num_subcores
    rows_per_w = t // (nc * ns)
    return _make_sc_scatter(t, vocab, rows_per_w, nc, ns)(cols, vals, g)


# speedup vs baseline: 19.7233x; 1.0347x over previous
import functools

import jax
import jax.numpy as jnp
from jax import lax
from jax.experimental import pallas as pl
from jax.experimental.pallas import tpu as pltpu
from jax.experimental.pallas import tpu_sc as plsc

LANES = 16
NBUF = 3


def _prep_kernel(ids_ref, lagl_ref, scale_ref, hidden_ref, w_ref, b_ref,
                 cols_ref, vals_ref, g_ref, *, window, vocab):
    h = hidden_ref[...][0]
    w = w_ref[...]
    logits = jnp.dot(h, w, preferred_element_type=jnp.float32)
    g = jax.nn.sigmoid(logits + b_ref[...][0, 0])[:, 0:1]
    g_ref[...] = jnp.broadcast_to(g, g_ref.shape)

    lw_row = (jax.nn.softmax(lagl_ref[...], axis=1)
              * scale_ref[...][0, 0])

    ids = ids_ref[...]
    t = ids.shape[1]
    shifted = [ids]
    for l in range(1, window):
        shifted.append(jnp.concatenate(
            [jnp.full((1, l), -1, jnp.int32), ids[:, : t - l]], axis=1))
    toks = jnp.concatenate(shifted, axis=0)

    merged = jnp.zeros((window, t), jnp.float32)
    notfirst = jnp.zeros((window, t), jnp.bool_)
    row = lax.broadcasted_iota(jnp.int32, (window, t), 0)
    for l2 in range(window):
        eq = toks == toks[l2:l2 + 1, :]
        merged = merged + jnp.where(eq, lw_row[0, l2], 0.0)
        if l2 < window - 1:
            notfirst = notfirst | (eq & (row > l2))
    keep = (toks >= 0) & jnp.logical_not(notfirst)
    cols8 = jnp.where(keep, toks, vocab + row)
    vals8 = jnp.where(keep, merged, 0.0)
    pad_cols = vocab + window + lax.broadcasted_iota(
        jnp.int32, (LANES - window, t), 0)
    cols_ref[...] = jnp.concatenate([cols8, pad_cols], axis=0)
    vals_ref[...] = jnp.concatenate(
        [vals8, jnp.zeros((LANES - window, t), jnp.float32)], axis=0)


def _make_sc_scatter(t, vocab, rows_per_w, nc, ns):
    bufw = vocab + 128
    zchunk = 8
    assert (vocab // LANES) % zchunk == 0
    mesh = plsc.VectorSubcoreMesh(core_axis_name="c", subcore_axis_name="s")

    @functools.partial(
        pl.kernel,
        out_type=jax.ShapeDtypeStruct((1, t, vocab), jnp.float32),
        mesh=mesh,
        scratch_types=[
            pltpu.VMEM((LANES, 2 * rows_per_w), jnp.int32),
            pltpu.VMEM((LANES, 2 * rows_per_w), jnp.float32),
            pltpu.VMEM((rows_per_w, 128), jnp.float32),
        ] + [pltpu.VMEM((bufw,), jnp.float32)] * NBUF
          + [pltpu.SemaphoreType.DMA] * (NBUF + 1),
        compiler_params=pltpu.CompilerParams(needs_layout_passes=False),
    )
    def sc_scatter(cols_hbm, vals_hbm, g_hbm, out_hbm, cols_v, vals_v, g_v,
                   *bufs_sems):
        bufs = bufs_sems[:NBUF]
        sems = bufs_sems[NBUF:]
        wid = lax.axis_index("s") * nc + lax.axis_index("c")
        base = wid * rows_per_w
        blk = (wid // 2) * (2 * rows_per_w)
        hoff = (wid % 2) * rows_per_w
        in_sem = bufs_sems[-1]
        cp_c = pltpu.async_copy(
            cols_hbm.at[:, pl.ds(blk, 2 * rows_per_w)], cols_v, in_sem)
        cp_v = pltpu.async_copy(
            vals_hbm.at[:, pl.ds(blk, 2 * rows_per_w)], vals_v, in_sem)
        cp_g = pltpu.async_copy(
            g_hbm.at[pl.ds(base, rows_per_w)], g_v, in_sem)

        z16 = jnp.zeros((LANES,), jnp.float32)
        z16i = jnp.zeros((LANES,), jnp.int32)
        iota16 = lax.broadcasted_iota(jnp.int32, (LANES,), 0)

        def zero_body(i, carry):
            for k in range(zchunk):
                off = (i * zchunk + k) * LANES
                for buf in bufs:
                    buf[pl.ds(off, LANES)] = z16
            return carry

        lax.fori_loop(0, vocab // LANES // zchunk, zero_body, 0)
        cp_c.wait()
        cp_v.wait()
        cp_g.wait()

        def row_vecs(r):
            rr = jnp.full((LANES,), r, jnp.int32) + hoff
            idx = plsc.load_gather(cols_v, (iota16, rr))
            return rr, idx

        copies = [None] * NBUF
        for r in range(rows_per_w):
            buf = bufs[r % NBUF]
            if r >= NBUF:
                copies[r % NBUF].wait()
                _, idx_old = row_vecs(r - NBUF)
                plsc.store_scatter(buf, (idx_old,), z16)
            rr, idx = row_vecs(r)
            vraw = plsc.load_gather(vals_v, (iota16, rr))
            gv = plsc.load_gather(g_v, (jnp.full((LANES,), r, jnp.int32), z16i))
            plsc.store_scatter(buf, (idx,), vraw * gv)
            copies[r % NBUF] = pltpu.async_copy(
                buf.at[pl.ds(0, vocab)],
                out_hbm.at[0, base + r],
                sems[r % NBUF])
        for k in range(NBUF):
            copies[k].wait()

    return sc_scatter


def kernel(hidden, input_ids, W, b_lin, lag_logits, copy_scale):
    b, t, d = hidden.shape
    vocab = 32000
    window = lag_logits.shape[0]
    lag_row = lag_logits.reshape(1, window)
    scale2 = copy_scale.reshape(1, 1)
    w_pad = jnp.pad(W.reshape(d, 1), ((0, 0), (0, 127)))
    b2 = b_lin.reshape(1, 1)

    cols, vals, g = pl.pallas_call(
        functools.partial(_prep_kernel, window=window, vocab=vocab),
        out_shape=[jax.ShapeDtypeStruct((LANES, t), jnp.int32),
                   jax.ShapeDtypeStruct((LANES, t), jnp.float32),
                   jax.ShapeDtypeStruct((t, 128), jnp.float32)],
    )(input_ids, lag_row, scale2, hidden, w_pad, b2)

    info = plsc.get_sparse_core_info()
    nc, ns = info.num_cores, info.num_subcores
    rows_per_w = t // (nc * ns)
    return _make_sc_scatter(t, vocab, rows_per_w, nc, ns)(cols, vals, g)


# speedup vs baseline: 20.0162x; 1.0149x over previous
import functools

import jax
import jax.numpy as jnp
from jax import lax
from jax.experimental import pallas as pl
from jax.experimental.pallas import tpu as pltpu
from jax.experimental.pallas import tpu_sc as plsc

LANES = 16
NBUF = 3


def _prep_kernel(ids_ref, lagl_ref, scale_ref, hidden_ref, w_ref, b_ref,
                 cols_ref, vals_ref, g_ref, *, window, vocab):
    h = hidden_ref[...][0]
    w = w_ref[...]
    logits = jnp.dot(h, w, preferred_element_type=jnp.float32)
    g = jax.nn.sigmoid(logits + b_ref[...][0, 0])[:, 0:1]
    g_ref[...] = jnp.broadcast_to(g, g_ref.shape)

    lw_row = (jax.nn.softmax(lagl_ref[...], axis=1)
              * scale_ref[...][0, 0])

    ids = ids_ref[...]
    t = ids.shape[1]
    shifted = [ids]
    for l in range(1, window):
        shifted.append(jnp.concatenate(
            [jnp.full((1, l), -1, jnp.int32), ids[:, : t - l]], axis=1))
    toks = jnp.concatenate(shifted, axis=0)

    merged = jnp.zeros((window, t), jnp.float32)
    notfirst = jnp.zeros((window, t), jnp.bool_)
    row = lax.broadcasted_iota(jnp.int32, (window, t), 0)
    for l2 in range(window):
        eq = toks == toks[l2:l2 + 1, :]
        merged = merged + jnp.where(eq, lw_row[0, l2], 0.0)
        if l2 < window - 1:
            notfirst = notfirst | (eq & (row > l2))
    keep = (toks >= 0) & jnp.logical_not(notfirst)
    cols8 = jnp.where(keep, toks, vocab + row)
    vals8 = jnp.where(keep, merged, 0.0)
    pad_cols = vocab + window + lax.broadcasted_iota(
        jnp.int32, (LANES - window, t), 0)
    cols_ref[...] = jnp.concatenate([cols8, pad_cols], axis=0)
    vals_ref[...] = jnp.concatenate(
        [vals8, jnp.zeros((LANES - window, t), jnp.float32)], axis=0)


def _make_sc_scatter(t, vocab, rows_per_w, nc, ns):
    bufw = vocab + 128
    zchunk = 8
    assert (vocab // LANES) % zchunk == 0
    mesh = plsc.VectorSubcoreMesh(core_axis_name="c", subcore_axis_name="s")

    @functools.partial(
        pl.kernel,
        out_type=jax.ShapeDtypeStruct((1, t, vocab), jnp.float32),
        mesh=mesh,
        scratch_types=[
            pltpu.VMEM((LANES, 2 * rows_per_w), jnp.int32),
            pltpu.VMEM((LANES, 2 * rows_per_w), jnp.float32),
            pltpu.VMEM((rows_per_w, 128), jnp.float32),
        ] + [pltpu.VMEM((bufw,), jnp.float32)] * NBUF
          + [pltpu.SemaphoreType.DMA] * (NBUF + 1),
        compiler_params=pltpu.CompilerParams(needs_layout_passes=False),
    )
    def sc_scatter(cols_hbm, vals_hbm, g_hbm, out_hbm, cols_v, vals_v, g_v,
                   *bufs_sems):
        bufs = bufs_sems[:NBUF]
        sems = bufs_sems[NBUF:]
        wid = lax.axis_index("s") * nc + lax.axis_index("c")
        base = wid * rows_per_w
        blk = (wid // 2) * (2 * rows_per_w)
        hoff = (wid % 2) * rows_per_w
        in_sem = bufs_sems[-1]
        cp_c = pltpu.async_copy(
            cols_hbm.at[:, pl.ds(blk, 2 * rows_per_w)], cols_v, in_sem)
        cp_v = pltpu.async_copy(
            vals_hbm.at[:, pl.ds(blk, 2 * rows_per_w)], vals_v, in_sem)
        cp_g = pltpu.async_copy(
            g_hbm.at[pl.ds(base, rows_per_w)], g_v, in_sem)

        z16 = jnp.zeros((LANES,), jnp.float32)
        z16i = jnp.zeros((LANES,), jnp.int32)
        iota16 = lax.broadcasted_iota(jnp.int32, (LANES,), 0)

        def zero_buf(buf):
            def zero_body(i, carry):
                for k in range(zchunk):
                    buf[pl.ds((i * zchunk + k) * LANES, LANES)] = z16
                return carry
            lax.fori_loop(0, vocab // LANES // zchunk, zero_body, 0)

        zero_buf(bufs[0])
        cp_c.wait()
        cp_v.wait()
        cp_g.wait()

        def row_vecs(r):
            rr = jnp.full((LANES,), r, jnp.int32) + hoff
            idx = plsc.load_gather(cols_v, (iota16, rr))
            return rr, idx

        copies = [None] * NBUF
        for r in range(rows_per_w):
            buf = bufs[r % NBUF]
            if 1 <= r < NBUF:
                zero_buf(buf)
            if r >= NBUF:
                copies[r % NBUF].wait()
                _, idx_old = row_vecs(r - NBUF)
                plsc.store_scatter(buf, (idx_old,), z16)
            rr, idx = row_vecs(r)
            vraw = plsc.load_gather(vals_v, (iota16, rr))
            gv = plsc.load_gather(g_v, (jnp.full((LANES,), r, jnp.int32), z16i))
            plsc.store_scatter(buf, (idx,), vraw * gv)
            copies[r % NBUF] = pltpu.async_copy(
                buf.at[pl.ds(0, vocab)],
                out_hbm.at[0, base + r],
                sems[r % NBUF])
        for k in range(NBUF):
            copies[k].wait()

    return sc_scatter


def kernel(hidden, input_ids, W, b_lin, lag_logits, copy_scale):
    b, t, d = hidden.shape
    vocab = 32000
    window = lag_logits.shape[0]
    lag_row = lag_logits.reshape(1, window)
    scale2 = copy_scale.reshape(1, 1)
    w_pad = jnp.pad(W.reshape(d, 1), ((0, 0), (0, 127)))
    b2 = b_lin.reshape(1, 1)

    cols, vals, g = pl.pallas_call(
        functools.partial(_prep_kernel, window=window, vocab=vocab),
        out_shape=[jax.ShapeDtypeStruct((LANES, t), jnp.int32),
                   jax.ShapeDtypeStruct((LANES, t), jnp.float32),
                   jax.ShapeDtypeStruct((t, 128), jnp.float32)],
    )(input_ids, lag_row, scale2, hidden, w_pad, b2)

    info = plsc.get_sparse_core_info()
    nc, ns = info.num_cores, info.num_subcores
    rows_per_w = t // (nc * ns)
    return _make_sc_scatter(t, vocab, rows_per_w, nc, ns)(cols, vals, g)
